# Initial kernel scaffold; baseline (speedup 1.0000x reference)
import jax
import jax.numpy as jnp
from jax.experimental import pallas as pl


def kernel(i_input, j_input, ks_input, user_embeddings, item_embeddings, time_embeddings):
    raise NotImplementedError("write your pallas kernel here")


# speedup vs baseline: 1.2881x; 1.2881x over previous
import functools

import jax
import jax.numpy as jnp
from jax import lax
from jax.experimental import pallas as pl
from jax.experimental.pallas import tpu as pltpu
from jax.experimental.pallas import tpu_sc as plsc

NC = 2
NS = 16
NW = NC * NS
CHUNK = 128


@functools.cache
def _build(B, HIST, D):
    n_ui = B // NW // CHUNK
    n_k = B * HIST // NW // CHUNK
    mesh = plsc.VectorSubcoreMesh(core_axis_name="c", subcore_axis_name="s")

    @functools.partial(
        pl.kernel,
        out_type=(
            jax.ShapeDtypeStruct((B, D), jnp.float32),
            jax.ShapeDtypeStruct((B, D), jnp.float32),
            jax.ShapeDtypeStruct((B * HIST, D), jnp.float32),
        ),
        mesh=mesh,
        compiler_params=pltpu.CompilerParams(use_tc_tiling_on_sc=False),
        scratch_types=[
            pltpu.VMEM((n_ui * CHUNK,), jnp.int32),
            pltpu.VMEM((n_ui * CHUNK,), jnp.int32),
            pltpu.VMEM((n_k * CHUNK,), jnp.int32),
            pltpu.VMEM((n_ui * CHUNK, D), jnp.float32),
            pltpu.VMEM((n_ui * CHUNK, D), jnp.float32),
            pltpu.VMEM((CHUNK, D), jnp.float32),
            pltpu.VMEM((CHUNK, D), jnp.float32),
            pltpu.SemaphoreType.DMA,
            pltpu.SemaphoreType.DMA,
            pltpu.SemaphoreType.DMA,
            pltpu.SemaphoreType.DMA,
        ],
    )
    def sc_kernel(i_hbm, j_hbm, ks_hbm, user_hbm, item_hbm, time_hbm,
                  out_i, out_j, out_k,
                  idx_u, idx_j, idx_k, rows_u, rows_j, buf0, buf1,
                  sem_u, sem_j, sem0, sem1):
        wid = lax.axis_index("s") * NC + lax.axis_index("c")
        ubase = pl.multiple_of(wid * (n_ui * CHUNK), CHUNK)
        kbase = pl.multiple_of(wid * (n_k * CHUNK), CHUNK)

        def kidx(chunk):
            return idx_k.at[pl.ds(pl.multiple_of(chunk * CHUNK, CHUNK), CHUNK)]

        pltpu.sync_copy(i_hbm.at[pl.ds(ubase, n_ui * CHUNK)], idx_u)
        cp_u = pltpu.make_async_copy(user_hbm.at[idx_u], rows_u, sem_u)
        cp_u.start()
        pltpu.sync_copy(j_hbm.at[pl.ds(ubase, n_ui * CHUNK)], idx_j)
        cp_j = pltpu.make_async_copy(item_hbm.at[idx_j], rows_j, sem_j)
        cp_j.start()

        pltpu.sync_copy(ks_hbm.at[pl.ds(kbase, n_k * CHUNK)], idx_k)
        bufs = (buf0, buf1)
        sems = (sem0, sem1)
        for b in range(2):
            pltpu.make_async_copy(
                time_hbm.at[kidx(b)], bufs[b], sems[b]).start()

        @pl.loop(0, n_k - 2, step=2)
        def _(c):
            for b in range(2):
                chunk = c + b
                pltpu.make_async_copy(
                    time_hbm.at[kidx(chunk)], bufs[b], sems[b]).wait()
                pltpu.sync_copy(
                    bufs[b],
                    out_k.at[pl.ds(kbase + pl.multiple_of(chunk * CHUNK, CHUNK),
                                   CHUNK)])
                pltpu.make_async_copy(
                    time_hbm.at[kidx(chunk + 2)], bufs[b], sems[b]).start()

        for b in range(2):
            chunk = n_k - 2 + b
            pltpu.make_async_copy(
                time_hbm.at[kidx(chunk)], bufs[b], sems[b]).wait()
            pltpu.sync_copy(
                bufs[b], out_k.at[pl.ds(kbase + chunk * CHUNK, CHUNK)])

        cp_u.wait()
        pltpu.sync_copy(rows_u, out_i.at[pl.ds(ubase, n_ui * CHUNK)])
        cp_j.wait()
        pltpu.sync_copy(rows_j, out_j.at[pl.ds(ubase, n_ui * CHUNK)])

    return sc_kernel


def kernel(i_input, j_input, ks_input, user_embeddings, item_embeddings,
           time_embeddings):
    B, HIST = ks_input.shape
    D = user_embeddings.shape[1]
    i2 = i_input.astype(jnp.int32)
    j2 = j_input.astype(jnp.int32)
    ks2 = ks_input.astype(jnp.int32).reshape(-1)
    sc = _build(B, HIST, D)
    out_i, out_j, out_k = sc(i2, j2, ks2, user_embeddings, item_embeddings,
                             time_embeddings)
    return (out_i, out_j, out_k.reshape(B, HIST, D))


# speedup vs baseline: 1.3318x; 1.0340x over previous
import functools

import jax
import jax.numpy as jnp
from jax import lax
from jax.experimental import pallas as pl
from jax.experimental.pallas import tpu as pltpu
from jax.experimental.pallas import tpu_sc as plsc

NC = 2
NS = 16
NW = NC * NS
CHUNK = 128


def _k_body(T, BT, ks_ref, table_ref, out_ref):
    idx = ks_ref[0, 0, :]
    rows = jax.lax.broadcasted_iota(jnp.int32, (T, BT), 0)
    onehot = jnp.where(rows == idx[None, :], 1.0, 0.0).astype(jnp.float32)
    out_ref[0] = jax.lax.dot_general(
        table_ref[...], onehot,
        dimension_numbers=(((1,), (0,)), ((), ())),
        precision=jax.lax.Precision.HIGHEST,
        preferred_element_type=jnp.float32)


@functools.cache
def _build_k(B, HIST, D, T, BT):
    grid = (HIST, B // BT)
    return pl.pallas_call(
        functools.partial(_k_body, T, BT),
        grid=grid,
        in_specs=[
            pl.BlockSpec((1, 1, BT), lambda h, b: (h, 0, b)),
            pl.BlockSpec((D, T), lambda h, b: (0, 0)),
        ],
        out_specs=pl.BlockSpec((1, D, BT), lambda h, b: (h, 0, b)),
        out_shape=jax.ShapeDtypeStruct((HIST, D, B), jnp.float32),
    )


@functools.cache
def _build_uij(B, D):
    n_ui = B // NW // CHUNK
    mesh = plsc.VectorSubcoreMesh(core_axis_name="c", subcore_axis_name="s")

    @functools.partial(
        pl.kernel,
        out_type=(
            jax.ShapeDtypeStruct((B, D), jnp.float32),
            jax.ShapeDtypeStruct((B, D), jnp.float32),
        ),
        mesh=mesh,
        compiler_params=pltpu.CompilerParams(use_tc_tiling_on_sc=False),
        scratch_types=[
            pltpu.VMEM((n_ui * CHUNK,), jnp.int32),
            pltpu.VMEM((n_ui * CHUNK,), jnp.int32),
            pltpu.VMEM((n_ui * CHUNK, D), jnp.float32),
            pltpu.VMEM((n_ui * CHUNK, D), jnp.float32),
            pltpu.SemaphoreType.DMA,
            pltpu.SemaphoreType.DMA,
        ],
    )
    def sc_kernel(i_hbm, j_hbm, user_hbm, item_hbm,
                  out_i, out_j,
                  idx_u, idx_j, rows_u, rows_j, sem_u, sem_j):
        wid = lax.axis_index("s") * NC + lax.axis_index("c")
        ubase = pl.multiple_of(wid * (n_ui * CHUNK), CHUNK)

        pltpu.sync_copy(i_hbm.at[pl.ds(ubase, n_ui * CHUNK)], idx_u)
        cp_u = pltpu.make_async_copy(user_hbm.at[idx_u], rows_u, sem_u)
        cp_u.start()
        pltpu.sync_copy(j_hbm.at[pl.ds(ubase, n_ui * CHUNK)], idx_j)
        cp_j = pltpu.make_async_copy(item_hbm.at[idx_j], rows_j, sem_j)
        cp_j.start()

        cp_u.wait()
        pltpu.sync_copy(rows_u, out_i.at[pl.ds(ubase, n_ui * CHUNK)])
        cp_j.wait()
        pltpu.sync_copy(rows_j, out_j.at[pl.ds(ubase, n_ui * CHUNK)])

    return sc_kernel


def kernel(i_input, j_input, ks_input, user_embeddings, item_embeddings,
           time_embeddings):
    B, HIST = ks_input.shape
    D = user_embeddings.shape[1]
    T = time_embeddings.shape[0]

    ks_t = ks_input.astype(jnp.int32).T.reshape(HIST, 1, B)
    time_t = time_embeddings.T
    out_k_t = _build_k(B, HIST, D, T, 512)(ks_t, time_t)
    out_k = jnp.transpose(out_k_t, (2, 0, 1))

    i2 = i_input.astype(jnp.int32)
    j2 = j_input.astype(jnp.int32)
    out_i, out_j = _build_uij(B, D)(i2, j2, user_embeddings, item_embeddings)
    return (out_i, out_j, out_k)


# speedup vs baseline: 1.5558x; 1.1682x over previous
import functools

import jax
import jax.numpy as jnp
from jax import lax
from jax.experimental import pallas as pl
from jax.experimental.pallas import tpu as pltpu
from jax.experimental.pallas import tpu_sc as plsc

NC = 2
NS = 16
NW = NC * NS
CHUNK = 128


def _k_body(T, ks_ref, table_ref, out_ref):
    idx = ks_ref[0, 0, :]
    rows = jax.lax.broadcasted_iota(jnp.int32, (T, idx.shape[0]), 0)
    onehot = jnp.where(rows == idx[None, :], 1.0, 0.0).astype(jnp.float32)
    out_ref[0] = jax.lax.dot_general(
        table_ref[...], onehot,
        dimension_numbers=(((1,), (0,)), ((), ())),
        precision=jax.lax.Precision.HIGHEST,
        preferred_element_type=jnp.float32)


@functools.cache
def _build_k(B, HIST, D, T, BT):
    grid = (HIST,)
    return pl.pallas_call(
        functools.partial(_k_body, T),
        grid=grid,
        in_specs=[
            pl.BlockSpec((1, 1, B), lambda h: (h, 0, 0)),
            pl.BlockSpec((D, T), lambda h: (0, 0)),
        ],
        out_specs=pl.BlockSpec((1, D, B), lambda h: (h, 0, 0)),
        out_shape=jax.ShapeDtypeStruct((HIST, D, B), jnp.float32),
    )


@functools.cache
def _build_uij(B, D):
    n_ui = B // NW // CHUNK
    mesh = plsc.VectorSubcoreMesh(core_axis_name="c", subcore_axis_name="s")

    @functools.partial(
        pl.kernel,
        out_type=(
            jax.ShapeDtypeStruct((B, D), jnp.float32),
            jax.ShapeDtypeStruct((B, D), jnp.float32),
        ),
        mesh=mesh,
        compiler_params=pltpu.CompilerParams(use_tc_tiling_on_sc=False),
        scratch_types=[
            pltpu.VMEM((n_ui * CHUNK,), jnp.int32),
            pltpu.VMEM((n_ui * CHUNK,), jnp.int32),
            pltpu.VMEM((n_ui * CHUNK, D), jnp.float32),
            pltpu.VMEM((n_ui * CHUNK, D), jnp.float32),
            pltpu.SemaphoreType.DMA,
            pltpu.SemaphoreType.DMA,
        ],
    )
    def sc_kernel(i_hbm, j_hbm, user_hbm, item_hbm,
                  out_i, out_j,
                  idx_u, idx_j, rows_u, rows_j, sem_u, sem_j):
        wid = lax.axis_index("s") * NC + lax.axis_index("c")
        ubase = pl.multiple_of(wid * (n_ui * CHUNK), CHUNK)

        pltpu.sync_copy(i_hbm.at[pl.ds(ubase, n_ui * CHUNK)], idx_u)
        cp_u = pltpu.make_async_copy(user_hbm.at[idx_u], rows_u, sem_u)
        cp_u.start()
        pltpu.sync_copy(j_hbm.at[pl.ds(ubase, n_ui * CHUNK)], idx_j)
        cp_j = pltpu.make_async_copy(item_hbm.at[idx_j], rows_j, sem_j)
        cp_j.start()

        cp_u.wait()
        pltpu.sync_copy(rows_u, out_i.at[pl.ds(ubase, n_ui * CHUNK)])
        cp_j.wait()
        pltpu.sync_copy(rows_j, out_j.at[pl.ds(ubase, n_ui * CHUNK)])

    return sc_kernel


def kernel(i_input, j_input, ks_input, user_embeddings, item_embeddings,
           time_embeddings):
    B, HIST = ks_input.shape
    D = user_embeddings.shape[1]
    T = time_embeddings.shape[0]

    ks_t = ks_input.astype(jnp.int32).T.reshape(HIST, 1, B)
    time_t = time_embeddings.T
    out_k_t = _build_k(B, HIST, D, T, 512)(ks_t, time_t)
    out_k = jnp.transpose(out_k_t, (2, 0, 1))

    i2 = i_input.astype(jnp.int32)
    j2 = j_input.astype(jnp.int32)
    out_i, out_j = _build_uij(B, D)(i2, j2, user_embeddings, item_embeddings)
    return (out_i, out_j, out_k)


# speedup vs baseline: 6.3536x; 4.0838x over previous
import functools

import jax
import jax.numpy as jnp
from jax import lax
from jax.experimental import pallas as pl
from jax.experimental.pallas import tpu as pltpu
from jax.experimental.pallas import tpu_sc as plsc

NC = 2
NS = 16
NW = NC * NS
LANES = 128


def _k_body(T, ks_ref, table_ref, out_ref):
    idx = ks_ref[0, 0, :]
    rows = jax.lax.broadcasted_iota(jnp.int32, (T, idx.shape[0]), 0)
    onehot = jnp.where(rows == idx[None, :], 1.0, 0.0).astype(jnp.float32)
    out_ref[0] = jax.lax.dot_general(
        table_ref[...], onehot,
        dimension_numbers=(((1,), (0,)), ((), ())),
        precision=jax.lax.Precision.HIGHEST,
        preferred_element_type=jnp.float32)


@functools.cache
def _build_k(B, HIST, D, T):
    return pl.pallas_call(
        functools.partial(_k_body, T),
        grid=(HIST,),
        in_specs=[
            pl.BlockSpec((1, 1, B), lambda h: (h, 0, 0)),
            pl.BlockSpec((D, T), lambda h: (0, 0)),
        ],
        out_specs=pl.BlockSpec((1, D, B), lambda h: (h, 0, 0)),
        out_shape=jax.ShapeDtypeStruct((HIST, D, B), jnp.float32),
    )


@functools.cache
def _build_uij(B, D, NU, NJ):
    per_tile = B // NW
    tsu = (NU // LANES - 1) * LANES
    tsj = (NJ // LANES - 1) * LANES
    mesh = plsc.VectorSubcoreMesh(core_axis_name="c", subcore_axis_name="s")

    @functools.partial(
        pl.kernel,
        out_type=(
            jax.ShapeDtypeStruct((D, B), jnp.float32),
            jax.ShapeDtypeStruct((D, B), jnp.float32),
        ),
        mesh=mesh,
        compiler_params=pltpu.CompilerParams(needs_layout_passes=False),
        scratch_types=[
            pltpu.VMEM((per_tile + 16,), jnp.int32),
            pltpu.VMEM((per_tile + 16,), jnp.int32),
            pltpu.VMEM((D, LANES), jnp.float32),
            pltpu.VMEM((D, LANES), jnp.float32),
            pltpu.VMEM((D, per_tile), jnp.float32),
            pltpu.SemaphoreType.DMA,
            pltpu.SemaphoreType.DMA,
        ],
    )
    def sc_kernel(i_hbm, j_hbm, user_t, item_t, tail_u, tail_j,
                  out_i, out_j,
                  sm_i, sm_j, buf0, buf1, outblk, sem0, sem1):
        wid = lax.axis_index("s") * NC + lax.axis_index("c")
        base = pl.multiple_of(wid * per_tile, per_tile)
        bufs = (buf0, buf1)
        sems = (sem0, sem1)
        row16 = [lax.broadcasted_iota(jnp.int32, (16,), 0) + 16 * r
                 for r in range(D // 16)]

        def run_table(idx_sm, tab, tail, ts, out_ref):
            def get(e):
                return idx_sm[pl.ds(e, 16)][0]

            def start(e, b):
                i = get(e)

                @pl.when(i < ts)
                def _():
                    c = pl.multiple_of((i >> 7) * LANES, LANES)
                    pltpu.make_async_copy(
                        tab.at[:, pl.ds(c, LANES)], bufs[b], sems[b]).start()

                @pl.when(i >= ts)
                def _():
                    c = pl.multiple_of(((i - ts) >> 7) * LANES, LANES)
                    pltpu.make_async_copy(
                        tail.at[:, pl.ds(c, LANES)], bufs[b], sems[b]).start()

            def finish(e, b):
                pltpu.make_async_copy(
                    tab.at[:, pl.ds(0, LANES)], bufs[b], sems[b]).wait()
                l16 = jnp.full((16,), get(e) & (LANES - 1), jnp.int32)
                e16 = jnp.full((16,), e, jnp.int32)
                for r in range(D // 16):
                    vals = plsc.load_gather(bufs[b], [row16[r], l16])
                    plsc.store_scatter(out_ref, [row16[r], e16], vals)

            start(0, 0)
            start(1, 1)

            @pl.loop(0, per_tile - 2, step=2)
            def _(e):
                for b in range(2):
                    finish(e + b, b)
                    start(e + b + 2, b)

            for b in range(2):
                finish(per_tile - 2 + b, b)

        pltpu.sync_copy(i_hbm.at[pl.ds(base, per_tile)],
                        sm_i.at[pl.ds(0, per_tile)])
        pltpu.sync_copy(j_hbm.at[pl.ds(base, per_tile)],
                        sm_j.at[pl.ds(0, per_tile)])

        run_table(sm_i, user_t, tail_u, tsu, outblk)
        pltpu.sync_copy(outblk, out_i.at[:, pl.ds(base, per_tile)])

        run_table(sm_j, item_t, tail_j, tsj, outblk)
        pltpu.sync_copy(outblk, out_j.at[:, pl.ds(base, per_tile)])

    return sc_kernel


def _tail(table_t, n):
    ts = (n // LANES - 1) * LANES
    return jnp.pad(table_t[:, ts:], ((0, 0), (0, 2 * LANES - (n - ts))))


def kernel(i_input, j_input, ks_input, user_embeddings, item_embeddings,
           time_embeddings):
    B, HIST = ks_input.shape
    D = user_embeddings.shape[1]
    T = time_embeddings.shape[0]
    NU = user_embeddings.shape[0]
    NJ = item_embeddings.shape[0]

    ks_t = ks_input.astype(jnp.int32).T.reshape(HIST, 1, B)
    time_t = time_embeddings.T
    out_k_t = _build_k(B, HIST, D, T)(ks_t, time_t)
    out_k = jnp.transpose(out_k_t, (2, 0, 1))

    user_t = user_embeddings.T
    item_t = item_embeddings.T
    out_i_t, out_j_t = _build_uij(B, D, NU, NJ)(
        i_input.astype(jnp.int32), j_input.astype(jnp.int32),
        user_t, item_t, _tail(user_t, NU), _tail(item_t, NJ))
    return (out_i_t.T, out_j_t.T, out_k)


# speedup vs baseline: 8.6992x; 1.3692x over previous
import functools

import jax
import jax.numpy as jnp
from jax import lax
from jax.experimental import pallas as pl
from jax.experimental.pallas import tpu as pltpu
from jax.experimental.pallas import tpu_sc as plsc

NC = 2
NS = 16
NW = NC * NS
LANES = 128


def _k_body(T, ks_ref, table_ref, out_ref):
    idx = ks_ref[0, 0, :]
    rows = jax.lax.broadcasted_iota(jnp.int32, (T, idx.shape[0]), 0)
    onehot = jnp.where(rows == idx[None, :], 1.0, 0.0).astype(jnp.float32)
    out_ref[0] = jax.lax.dot_general(
        table_ref[...], onehot,
        dimension_numbers=(((1,), (0,)), ((), ())),
        precision=jax.lax.Precision.HIGHEST,
        preferred_element_type=jnp.float32)


@functools.cache
def _build_k(B, HIST, D, T):
    return pl.pallas_call(
        functools.partial(_k_body, T),
        grid=(HIST,),
        in_specs=[
            pl.BlockSpec((1, 1, B), lambda h: (h, 0, 0)),
            pl.BlockSpec((D, T), lambda h: (0, 0)),
        ],
        out_specs=pl.BlockSpec((1, D, B), lambda h: (h, 0, 0)),
        out_shape=jax.ShapeDtypeStruct((HIST, D, B), jnp.float32),
    )


@functools.cache
def _build_uij(B, D, NU, NJ):
    per_tile = B // NW
    tsu = (NU // LANES - 1) * LANES
    tsj = (NJ // LANES - 1) * LANES
    mesh = plsc.VectorSubcoreMesh(core_axis_name="c", subcore_axis_name="s")

    @functools.partial(
        pl.kernel,
        out_type=(
            jax.ShapeDtypeStruct((D, B), jnp.float32),
            jax.ShapeDtypeStruct((D, B), jnp.float32),
        ),
        mesh=mesh,
        compiler_params=pltpu.CompilerParams(needs_layout_passes=False),
        scratch_types=[
            pltpu.VMEM((per_tile + 16,), jnp.int32),
            pltpu.VMEM((per_tile + 16,), jnp.int32),
            pltpu.VMEM((D, LANES), jnp.float32),
            pltpu.VMEM((D, LANES), jnp.float32),
            pltpu.VMEM((D, LANES), jnp.float32),
            pltpu.VMEM((D, LANES), jnp.float32),
            pltpu.VMEM((D, per_tile), jnp.float32),
            pltpu.VMEM((D, per_tile), jnp.float32),
            pltpu.SemaphoreType.DMA,
            pltpu.SemaphoreType.DMA,
            pltpu.SemaphoreType.DMA,
            pltpu.SemaphoreType.DMA,
        ],
    )
    def sc_kernel(i_hbm, j_hbm, user_t, item_t, tail_u, tail_j,
                  out_i, out_j,
                  sm_i, sm_j, bu0, bu1, bj0, bj1, oblk_u, oblk_j,
                  su0, su1, sj0, sj1):
        wid = lax.axis_index("s") * NC + lax.axis_index("c")
        base = pl.multiple_of(wid * per_tile, per_tile)
        row16 = [lax.broadcasted_iota(jnp.int32, (16,), 0) + 16 * r
                 for r in range(D // 16)]

        def make_ops(idx_sm, tab, tail, ts, out_ref, bufs, sems):
            def get(e):
                return idx_sm[pl.ds(e, 16)][0]

            def start(e, b):
                i = get(e)

                @pl.when(i < ts)
                def _():
                    c = pl.multiple_of((i >> 7) * LANES, LANES)
                    pltpu.make_async_copy(
                        tab.at[:, pl.ds(c, LANES)], bufs[b], sems[b]).start()

                @pl.when(i >= ts)
                def _():
                    c = pl.multiple_of(((i - ts) >> 7) * LANES, LANES)
                    pltpu.make_async_copy(
                        tail.at[:, pl.ds(c, LANES)], bufs[b], sems[b]).start()

            def finish(e, b):
                pltpu.make_async_copy(
                    tab.at[:, pl.ds(0, LANES)], bufs[b], sems[b]).wait()
                l16 = jnp.full((16,), get(e) & (LANES - 1), jnp.int32)
                e16 = jnp.full((16,), e, jnp.int32)
                for r in range(D // 16):
                    vals = plsc.load_gather(bufs[b], [row16[r], l16])
                    plsc.store_scatter(out_ref, [row16[r], e16], vals)

            return start, finish

        pltpu.sync_copy(i_hbm.at[pl.ds(base, per_tile)],
                        sm_i.at[pl.ds(0, per_tile)])
        pltpu.sync_copy(j_hbm.at[pl.ds(base, per_tile)],
                        sm_j.at[pl.ds(0, per_tile)])

        start_u, finish_u = make_ops(sm_i, user_t, tail_u, tsu, oblk_u,
                                     (bu0, bu1), (su0, su1))
        start_j, finish_j = make_ops(sm_j, item_t, tail_j, tsj, oblk_j,
                                     (bj0, bj1), (sj0, sj1))

        for b in range(2):
            start_u(b, b)
            start_j(b, b)

        @pl.loop(0, per_tile - 2, step=2)
        def _(e):
            for b in range(2):
                finish_u(e + b, b)
                start_u(e + b + 2, b)
                finish_j(e + b, b)
                start_j(e + b + 2, b)

        for b in range(2):
            finish_u(per_tile - 2 + b, b)
            finish_j(per_tile - 2 + b, b)

        pltpu.sync_copy(oblk_u, out_i.at[:, pl.ds(base, per_tile)])
        pltpu.sync_copy(oblk_j, out_j.at[:, pl.ds(base, per_tile)])

    return sc_kernel


def _tail(table_t, n):
    ts = (n // LANES - 1) * LANES
    return jnp.pad(table_t[:, ts:], ((0, 0), (0, 2 * LANES - (n - ts))))


def kernel(i_input, j_input, ks_input, user_embeddings, item_embeddings,
           time_embeddings):
    B, HIST = ks_input.shape
    D = user_embeddings.shape[1]
    T = time_embeddings.shape[0]
    NU = user_embeddings.shape[0]
    NJ = item_embeddings.shape[0]

    ks_t = ks_input.astype(jnp.int32).T.reshape(HIST, 1, B)
    time_t = time_embeddings.T
    out_k_t = _build_k(B, HIST, D, T)(ks_t, time_t)
    out_k = jnp.transpose(out_k_t, (2, 0, 1))

    user_t = user_embeddings.T
    item_t = item_embeddings.T
    out_i_t, out_j_t = _build_uij(B, D, NU, NJ)(
        i_input.astype(jnp.int32), j_input.astype(jnp.int32),
        user_t, item_t, _tail(user_t, NU), _tail(item_t, NJ))
    return (out_i_t.T, out_j_t.T, out_k)


# speedup vs baseline: 9.1003x; 1.0461x over previous
import functools

import jax
import jax.numpy as jnp
from jax import lax
from jax.experimental import pallas as pl
from jax.experimental.pallas import tpu as pltpu
from jax.experimental.pallas import tpu_sc as plsc

NC = 2
NS = 16
NW = NC * NS
LANES = 128
NBUF = 4


def _k_body(T, ks_ref, table_ref, out_ref):
    idx = ks_ref[0, 0, :]
    rows = jax.lax.broadcasted_iota(jnp.int32, (T, idx.shape[0]), 0)
    onehot = jnp.where(rows == idx[None, :], 1.0, 0.0).astype(jnp.float32)
    out_ref[0] = jax.lax.dot_general(
        table_ref[...], onehot,
        dimension_numbers=(((1,), (0,)), ((), ())),
        precision=jax.lax.Precision.HIGHEST,
        preferred_element_type=jnp.float32)


@functools.cache
def _build_k(B, HIST, D, T):
    return pl.pallas_call(
        functools.partial(_k_body, T),
        grid=(HIST,),
        in_specs=[
            pl.BlockSpec((1, 1, B), lambda h: (h, 0, 0)),
            pl.BlockSpec((D, T), lambda h: (0, 0)),
        ],
        out_specs=pl.BlockSpec((1, D, B), lambda h: (h, 0, 0)),
        out_shape=jax.ShapeDtypeStruct((HIST, D, B), jnp.float32),
    )


@functools.cache
def _build_uij(B, D, NU, NJ):
    per_tile = B // NW
    tsu = (NU // LANES - 1) * LANES
    tsj = (NJ // LANES - 1) * LANES
    mesh = plsc.VectorSubcoreMesh(core_axis_name="c", subcore_axis_name="s")

    @functools.partial(
        pl.kernel,
        out_type=(
            jax.ShapeDtypeStruct((D, B), jnp.float32),
            jax.ShapeDtypeStruct((D, B), jnp.float32),
        ),
        mesh=mesh,
        compiler_params=pltpu.CompilerParams(needs_layout_passes=False),
        scratch_types=[
            pltpu.VMEM((per_tile + 16,), jnp.int32),
            pltpu.VMEM((per_tile + 16,), jnp.int32),
        ] + [pltpu.VMEM((D, LANES), jnp.float32)] * (2 * NBUF)
          + [pltpu.VMEM((D, per_tile), jnp.float32)] * 2
          + [pltpu.SemaphoreType.DMA] * (2 * NBUF),
    )
    def sc_kernel(i_hbm, j_hbm, user_t, item_t, tail_u, tail_j,
                  out_i, out_j, sm_i, sm_j, *rest):
        bufs_u = rest[0:NBUF]
        bufs_j = rest[NBUF:2 * NBUF]
        oblk_u, oblk_j = rest[2 * NBUF:2 * NBUF + 2]
        sems_u = rest[2 * NBUF + 2:3 * NBUF + 2]
        sems_j = rest[3 * NBUF + 2:4 * NBUF + 2]
        wid = lax.axis_index("s") * NC + lax.axis_index("c")
        base = pl.multiple_of(wid * per_tile, per_tile)
        row16 = [lax.broadcasted_iota(jnp.int32, (16,), 0) + 16 * r
                 for r in range(D // 16)]

        def make_ops(idx_sm, tab, tail, ts, out_ref, bufs, sems):
            def get(e):
                return idx_sm[pl.ds(e, 16)][0]

            def start(e, b):
                i = get(e)

                @pl.when(i < ts)
                def _():
                    c = pl.multiple_of((i >> 7) * LANES, LANES)
                    pltpu.make_async_copy(
                        tab.at[:, pl.ds(c, LANES)], bufs[b], sems[b]).start()

                @pl.when(i >= ts)
                def _():
                    c = pl.multiple_of(((i - ts) >> 7) * LANES, LANES)
                    pltpu.make_async_copy(
                        tail.at[:, pl.ds(c, LANES)], bufs[b], sems[b]).start()

            def finish(e, b):
                pltpu.make_async_copy(
                    tab.at[:, pl.ds(0, LANES)], bufs[b], sems[b]).wait()
                l16 = jnp.full((16,), get(e) & (LANES - 1), jnp.int32)
                e16 = jnp.full((16,), e, jnp.int32)
                for r in range(D // 16):
                    vals = plsc.load_gather(bufs[b], [row16[r], l16])
                    plsc.store_scatter(out_ref, [row16[r], e16], vals)

            return start, finish

        pltpu.sync_copy(i_hbm.at[pl.ds(base, per_tile)],
                        sm_i.at[pl.ds(0, per_tile)])
        pltpu.sync_copy(j_hbm.at[pl.ds(base, per_tile)],
                        sm_j.at[pl.ds(0, per_tile)])

        start_u, finish_u = make_ops(sm_i, user_t, tail_u, tsu, oblk_u,
                                     bufs_u, sems_u)
        start_j, finish_j = make_ops(sm_j, item_t, tail_j, tsj, oblk_j,
                                     bufs_j, sems_j)

        for b in range(NBUF):
            start_u(b, b)
            start_j(b, b)

        @pl.loop(0, per_tile - NBUF, step=NBUF)
        def _(e):
            for b in range(NBUF):
                finish_u(e + b, b)
                start_u(e + b + NBUF, b)
                finish_j(e + b, b)
                start_j(e + b + NBUF, b)

        for b in range(NBUF):
            finish_u(per_tile - NBUF + b, b)
            finish_j(per_tile - NBUF + b, b)

        pltpu.sync_copy(oblk_u, out_i.at[:, pl.ds(base, per_tile)])
        pltpu.sync_copy(oblk_j, out_j.at[:, pl.ds(base, per_tile)])

    return sc_kernel


def _tail(table_t, n):
    ts = (n // LANES - 1) * LANES
    return jnp.pad(table_t[:, ts:], ((0, 0), (0, 2 * LANES - (n - ts))))


def kernel(i_input, j_input, ks_input, user_embeddings, item_embeddings,
           time_embeddings):
    B, HIST = ks_input.shape
    D = user_embeddings.shape[1]
    T = time_embeddings.shape[0]
    NU = user_embeddings.shape[0]
    NJ = item_embeddings.shape[0]

    ks_t = ks_input.astype(jnp.int32).T.reshape(HIST, 1, B)
    time_t = time_embeddings.T
    out_k_t = _build_k(B, HIST, D, T)(ks_t, time_t)
    out_k = jnp.transpose(out_k_t, (2, 0, 1))

    user_t = user_embeddings.T
    item_t = item_embeddings.T
    out_i_t, out_j_t = _build_uij(B, D, NU, NJ)(
        i_input.astype(jnp.int32), j_input.astype(jnp.int32),
        user_t, item_t, _tail(user_t, NU), _tail(item_t, NJ))
    return (out_i_t.T, out_j_t.T, out_k)


# speedup vs baseline: 9.1053x; 1.0006x over previous
import functools

import jax
import jax.numpy as jnp
from jax import lax
from jax.experimental import pallas as pl
from jax.experimental.pallas import tpu as pltpu
from jax.experimental.pallas import tpu_sc as plsc

NC = 2
NS = 16
NW = NC * NS
LANES = 128
NBUF = 4


def _k_body(T, ks_ref, table_ref, out_ref):
    idx = ks_ref[0, 0, :]
    rows = jax.lax.broadcasted_iota(jnp.int32, (T, idx.shape[0]), 0)
    onehot = jnp.where(rows == idx[None, :], 1.0, 0.0).astype(jnp.float32)
    out_ref[0] = jax.lax.dot_general(
        table_ref[...], onehot,
        dimension_numbers=(((1,), (0,)), ((), ())),
        precision=jax.lax.Precision.HIGHEST,
        preferred_element_type=jnp.float32)


@functools.cache
def _build_k(B, HIST, D, T):
    return pl.pallas_call(
        functools.partial(_k_body, T),
        grid=(HIST,),
        in_specs=[
            pl.BlockSpec((1, 1, B), lambda h: (h, 0, 0)),
            pl.BlockSpec((D, T), lambda h: (0, 0)),
        ],
        out_specs=pl.BlockSpec((1, D, B), lambda h: (h, 0, 0)),
        out_shape=jax.ShapeDtypeStruct((HIST, D, B), jnp.float32),
    )


@functools.cache
def _build_uij(B, D, NU, NJ):
    per_tile = B // NW
    tsu = (NU // LANES - 1) * LANES
    tsj = (NJ // LANES - 1) * LANES
    mesh = plsc.VectorSubcoreMesh(core_axis_name="c", subcore_axis_name="s")

    @functools.partial(
        pl.kernel,
        out_type=(
            jax.ShapeDtypeStruct((D, B), jnp.float32),
            jax.ShapeDtypeStruct((D, B), jnp.float32),
        ),
        mesh=mesh,
        compiler_params=pltpu.CompilerParams(needs_layout_passes=False),
        scratch_types=[
            pltpu.VMEM((per_tile + 16,), jnp.int32),
            pltpu.VMEM((per_tile + 16,), jnp.int32),
        ] + [pltpu.VMEM((D, LANES), jnp.float32)] * (2 * NBUF)
          + [pltpu.VMEM((D, per_tile), jnp.float32)] * 2
          + [pltpu.SemaphoreType.DMA] * (2 * NBUF),
    )
    def sc_kernel(i_hbm, j_hbm, user_t, item_t, tail_u, tail_j,
                  out_i, out_j, sm_i, sm_j, *rest):
        bufs_u = rest[0:NBUF]
        bufs_j = rest[NBUF:2 * NBUF]
        oblk_u, oblk_j = rest[2 * NBUF:2 * NBUF + 2]
        sems_u = rest[2 * NBUF + 2:3 * NBUF + 2]
        sems_j = rest[3 * NBUF + 2:4 * NBUF + 2]
        wid = lax.axis_index("s") * NC + lax.axis_index("c")
        base = pl.multiple_of(wid * per_tile, per_tile)
        row16 = [lax.broadcasted_iota(jnp.int32, (16,), 0) + 16 * r
                 for r in range(D // 16)]

        def make_ops(idx_sm, tab, tail, ts, out_ref, bufs, sems):
            def get(e):
                return idx_sm[pl.ds(e, 16)][0]

            def start(e, b):
                i = get(e)

                @pl.when(i < ts)
                def _():
                    c = pl.multiple_of((i >> 7) * LANES, LANES)
                    pltpu.make_async_copy(
                        tab.at[:, pl.ds(c, LANES)], bufs[b], sems[b]).start()

                @pl.when(i >= ts)
                def _():
                    c = pl.multiple_of(((i - ts) >> 7) * LANES, LANES)
                    pltpu.make_async_copy(
                        tail.at[:, pl.ds(c, LANES)], bufs[b], sems[b]).start()

            def finish(e, b):
                pltpu.make_async_copy(
                    tab.at[:, pl.ds(0, LANES)], bufs[b], sems[b]).wait()
                l16 = jnp.full((16,), get(e) & (LANES - 1), jnp.int32)
                e16 = jnp.full((16,), e, jnp.int32)
                for r in range(D // 16):
                    vals = plsc.load_gather(bufs[b], [row16[r], l16])
                    plsc.store_scatter(out_ref, [row16[r], e16], vals)

            return start, finish

        pltpu.sync_copy(i_hbm.at[pl.ds(base, per_tile)],
                        sm_i.at[pl.ds(0, per_tile)])
        pltpu.sync_copy(j_hbm.at[pl.ds(base, per_tile)],
                        sm_j.at[pl.ds(0, per_tile)])

        start_u, finish_u = make_ops(sm_i, user_t, tail_u, tsu, oblk_u,
                                     bufs_u, sems_u)
        start_j, finish_j = make_ops(sm_j, item_t, tail_j, tsj, oblk_j,
                                     bufs_j, sems_j)

        for b in range(NBUF):
            start_u(b, b)
            start_j(b, b)

        @pl.loop(0, per_tile - NBUF, step=NBUF)
        def _(e):
            for b in range(NBUF):
                finish_u(e + b, b)
                start_u(e + b + NBUF, b)
                finish_j(e + b, b)
                start_j(e + b + NBUF, b)

        for b in range(NBUF):
            finish_u(per_tile - NBUF + b, b)
            finish_j(per_tile - NBUF + b, b)

        pltpu.sync_copy(oblk_u, out_i.at[:, pl.ds(base, per_tile)])
        pltpu.sync_copy(oblk_j, out_j.at[:, pl.ds(base, per_tile)])

    return sc_kernel


def _tail(table_t, n):
    ts = (n // LANES - 1) * LANES
    return jnp.pad(table_t[:, ts:], ((0, 0), (0, 2 * LANES - (n - ts))))


def kernel(i_input, j_input, ks_input, user_embeddings, item_embeddings,
           time_embeddings):
    B, HIST = ks_input.shape
    D = user_embeddings.shape[1]
    T = time_embeddings.shape[0]
    NU = user_embeddings.shape[0]
    NJ = item_embeddings.shape[0]

    user_t = user_embeddings.T
    item_t = item_embeddings.T
    out_i_t, out_j_t = _build_uij(B, D, NU, NJ)(
        i_input.astype(jnp.int32), j_input.astype(jnp.int32),
        user_t, item_t, _tail(user_t, NU), _tail(item_t, NJ))

    ks_t = ks_input.astype(jnp.int32).T.reshape(HIST, 1, B)
    time_t = time_embeddings.T
    out_k_t = _build_k(B, HIST, D, T)(ks_t, time_t)
    out_k = jnp.transpose(out_k_t, (2, 0, 1))
    return (out_i_t.T, out_j_t.T, out_k)
